# baseline (speedup 1.0000x reference)
import jax
import jax.numpy as jnp
from jax.experimental import pallas as pl
from jax.experimental.pallas import tpu as pltpu

LANES = 8192
ROWS = 24
NBUF = 8


def _pos_body(xe_ref, ye_ref, pos_ref):
    pos_ref[...] = xe_ref[...][:, None, :] + ye_ref[...][None, :, :]


def _add_body(x_ref, pos_ref, o_ref, ibuf, obuf, isem, osem):
    i = pl.program_id(0)
    nsteps = pl.num_programs(0)
    slot = jax.lax.rem(i, NBUF)

    @pl.when(i == 0)
    def _prologue():
        for j in range(NBUF):
            pltpu.make_async_copy(
                x_ref.at[pl.ds(j * ROWS, ROWS)], ibuf.at[j], isem.at[j]
            ).start()

    pltpu.make_async_copy(
        x_ref.at[pl.ds(i * ROWS, ROWS)], ibuf.at[slot], isem.at[slot]
    ).wait()

    @pl.when(i >= NBUF)
    def _wait_prev_out():
        pltpu.make_async_copy(
            obuf.at[slot], o_ref.at[pl.ds((i - NBUF) * ROWS, ROWS)], osem.at[slot]
        ).wait()

    obuf[slot] = ibuf[slot] + pos_ref[...]

    pltpu.make_async_copy(
        obuf.at[slot], o_ref.at[pl.ds(i * ROWS, ROWS)], osem.at[slot]
    ).start()

    @pl.when(i + NBUF < nsteps)
    def _prefetch():
        pltpu.make_async_copy(
            x_ref.at[pl.ds((i + NBUF) * ROWS, ROWS)], ibuf.at[slot], isem.at[slot]
        ).start()

    @pl.when(i == nsteps - 1)
    def _drain():
        for j in range(NBUF):
            pltpu.make_async_copy(
                obuf.at[j], o_ref.at[pl.ds(j * ROWS, ROWS)], osem.at[j]
            ).wait()


def kernel(x, xemb, yemb):
    B, H, W, D = x.shape

    posemb = pl.pallas_call(
        _pos_body,
        out_shape=jax.ShapeDtypeStruct((H, W, D), x.dtype),
    )(xemb, yemb)

    pos2 = posemb.reshape(ROWS, LANES)
    x2 = x.reshape(B * ROWS, LANES)

    out = pl.pallas_call(
        _add_body,
        grid=(B,),
        in_specs=[
            pl.BlockSpec(memory_space=pltpu.MemorySpace.HBM),
            pl.BlockSpec((ROWS, LANES), lambda i: (0, 0)),
        ],
        out_specs=pl.BlockSpec(memory_space=pltpu.MemorySpace.HBM),
        out_shape=jax.ShapeDtypeStruct((B * ROWS, LANES), x.dtype),
        scratch_shapes=[
            pltpu.VMEM((NBUF, ROWS, LANES), x.dtype),
            pltpu.VMEM((NBUF, ROWS, LANES), x.dtype),
            pltpu.SemaphoreType.DMA((NBUF,)),
            pltpu.SemaphoreType.DMA((NBUF,)),
        ],
        compiler_params=pltpu.CompilerParams(
            dimension_semantics=("arbitrary",),
        ),
    )(x2, pos2)
    return out.reshape(B, H, W, D)
